# Initial kernel scaffold; baseline (speedup 1.0000x reference)
import functools

import jax
import jax.numpy as jnp
from jax import lax
from jax.experimental import pallas as pl
from jax.experimental.pallas import tpu as pltpu
from jax.experimental.pallas import tpu_sc as plsc

N = 10000
E = 160000
IN = 64
HID = 32
EDGE = 16
OUT = 128
G = 64

NC = 2
NS = 16
NW = NC * NS
PER_W = E // NW
CHUNK = 1000
NCHUNK = PER_W // CHUNK


def _sc_mesh():
    return plsc.VectorSubcoreMesh(core_axis_name="c", subcore_axis_name="s")


def _gather_body(x_hbm, src_hbm, out_hbm, idx_v, rows_v, sem):
    wid = lax.axis_index("s") * NC + lax.axis_index("c")
    base = wid * PER_W
    for c in range(NCHUNK):
        off = base + c * CHUNK
        pltpu.sync_copy(src_hbm.at[pl.ds(off, CHUNK)], idx_v)
        pltpu.async_copy(x_hbm.at[idx_v], rows_v, sem).wait()
        pltpu.sync_copy(rows_v, out_hbm.at[pl.ds(off, CHUNK)])


_sc_gather = functools.partial(
    pl.kernel,
    out_type=jax.ShapeDtypeStruct((E, IN), jnp.float32),
    mesh=_sc_mesh(),
    scratch_types=[
        pltpu.VMEM((CHUNK,), jnp.int32),
        pltpu.VMEM((CHUNK, IN), jnp.float32),
        pltpu.SemaphoreType.DMA,
    ],
)(_gather_body)


ROWS_PER_TILE = N // NS


def _scatter_body(msg_hbm, dst_hbm, zero_hbm, out_hbm, idx_v, msg_v, agg_sh):
    cid = lax.axis_index("c")
    sid = lax.axis_index("s")
    wid = sid * NC + cid
    r0 = sid * ROWS_PER_TILE
    pltpu.sync_copy(zero_hbm.at[pl.ds(r0, ROWS_PER_TILE)],
                    agg_sh.at[pl.ds(r0, ROWS_PER_TILE)])
    plsc.subcore_barrier()
    base = wid * PER_W
    for c in range(NCHUNK):
        off = base + c * CHUNK
        pltpu.sync_copy(dst_hbm.at[pl.ds(off, CHUNK)], idx_v)
        pltpu.sync_copy(msg_hbm.at[pl.ds(off, CHUNK)], msg_v)
        pltpu.sync_copy(msg_v, agg_sh.at[idx_v], add=True)
    plsc.subcore_barrier()
    pltpu.sync_copy(agg_sh.at[pl.ds(r0, ROWS_PER_TILE)],
                    out_hbm.at[cid, pl.ds(r0, ROWS_PER_TILE)])


_sc_scatter = functools.partial(
    pl.kernel,
    out_type=jax.ShapeDtypeStruct((NC, N, HID), jnp.float32),
    mesh=_sc_mesh(),
    scratch_types=[
        pltpu.VMEM((CHUNK,), jnp.int32),
        pltpu.VMEM((CHUNK, HID), jnp.float32),
        pltpu.VMEM_SHARED((N, HID), jnp.float32),
    ],
)(_scatter_body)


TE = 2000


def _msg_body(xj_ref, ea_ref, a2_ref, b_ref, out_ref):
    xj = xj_ref[...]
    p = jnp.dot(xj, a2_ref[...], preferred_element_type=jnp.float32)
    acc = jnp.dot(xj, b_ref[...], preferred_element_type=jnp.float32)
    ea = ea_ref[...]
    for k in range(EDGE):
        acc = acc + ea[:, k:k + 1] * p[:, k * HID:(k + 1) * HID]
    out_ref[...] = acc


def _msg_kernel(x_j, ea, a2, bmat):
    return pl.pallas_call(
        _msg_body,
        grid=(E // TE,),
        in_specs=[
            pl.BlockSpec((TE, IN), lambda i: (i, 0)),
            pl.BlockSpec((TE, EDGE), lambda i: (i, 0)),
            pl.BlockSpec((IN, EDGE * HID), lambda i: (0, 0)),
            pl.BlockSpec((IN, HID), lambda i: (0, 0)),
        ],
        out_specs=pl.BlockSpec((TE, HID), lambda i: (i, 0)),
        out_shape=jax.ShapeDtypeStruct((E, HID), jnp.float32),
        compiler_params=pltpu.CompilerParams(
            dimension_semantics=("arbitrary",)),
    )(x_j, ea, a2, bmat)


def _tail_body(agg_ref, x_ref, root_ref, cb_ref, g_ref, be_ref,
               sw_ref, sb_ref, tw_ref, tb_ref, batch_ref, out_ref):
    h = (agg_ref[0] + agg_ref[1]
         + jnp.dot(x_ref[...], root_ref[...], preferred_element_type=jnp.float32)
         + cb_ref[...])
    mean = jnp.mean(h, axis=0, keepdims=True)
    var = jnp.mean((h - mean) * (h - mean), axis=0, keepdims=True)
    hn = (h - mean) * lax.rsqrt(var + 1e-5) * g_ref[...] + be_ref[...]
    hr = jnp.maximum(hn, 0.0)
    s_x = jnp.dot(hr, sw_ref[...], preferred_element_type=jnp.float32) + sb_ref[...]
    t_x = jnp.dot(hr, tw_ref[...], preferred_element_type=jnp.float32) + tb_ref[...]
    s_x = jnp.clip(s_x, -30.0, 30.0)
    f_x = 1.0 / (1.0 + jnp.exp(s_x)) * jnp.tanh(t_x)
    gids = lax.broadcasted_iota(jnp.float32, (G, N), 0)
    onehot = jnp.where(gids == batch_ref[...], 1.0, 0.0)
    out_ref[...] = jnp.dot(onehot, f_x, preferred_element_type=jnp.float32)


def _tail_kernel(agg, x, root, conv_bias, bn_gamma, bn_beta,
                 s_wt, s_b, t_wt, t_b, batchf):
    return pl.pallas_call(
        _tail_body,
        out_shape=jax.ShapeDtypeStruct((G, OUT), jnp.float32),
    )(agg, x, root, conv_bias, bn_gamma, bn_beta,
      s_wt, s_b, t_wt, t_b, batchf)


def kernel(x, edge_index, edge_attr, batch, edge_batch, nn_W, nn_b, root,
           conv_bias, bn_gamma, bn_beta, s_W, s_b, t_W, t_b):
    src = edge_index[0].astype(jnp.int32)
    dst = edge_index[1].astype(jnp.int32)
    a2 = nn_W.reshape(IN, HID, EDGE).transpose(0, 2, 1).reshape(IN, EDGE * HID)
    bmat = nn_b.reshape(IN, HID)

    x_j = _sc_gather(x, src)
    msg = _msg_kernel(x_j, edge_attr, a2, bmat)
    zeros = jnp.zeros((N, HID), jnp.float32)
    agg = _sc_scatter(msg, dst, zeros)

    feature = _tail_kernel(
        agg, x, root,
        conv_bias.reshape(1, HID), bn_gamma.reshape(1, HID),
        bn_beta.reshape(1, HID),
        s_W.T, s_b.reshape(1, OUT), t_W.T, t_b.reshape(1, OUT),
        batch.astype(jnp.float32).reshape(1, N))
    return feature


# speedup vs baseline: 2.3727x; 2.3727x over previous
import functools

import jax
import jax.numpy as jnp
from jax import lax
from jax.experimental import pallas as pl
from jax.experimental.pallas import tpu as pltpu
from jax.experimental.pallas import tpu_sc as plsc

N = 10000
E = 160000
IN = 64
HID = 32
EDGE = 16
OUT = 128
G = 64

NC = 2
NS = 16
NW = NC * NS
PER_W = E // NW
CHUNK = 1000
NCHUNK = PER_W // CHUNK


def _sc_mesh():
    return plsc.VectorSubcoreMesh(core_axis_name="c", subcore_axis_name="s")


def _gather_body(x_hbm, src_hbm, out_hbm, idx_v, rows_v, sem):
    wid = lax.axis_index("s") * NC + lax.axis_index("c")
    base = wid * PER_W
    for c in range(NCHUNK):
        off = base + c * CHUNK
        pltpu.sync_copy(src_hbm.at[pl.ds(off, CHUNK)], idx_v)
        pltpu.async_copy(x_hbm.at[idx_v], rows_v, sem).wait()
        pltpu.sync_copy(rows_v, out_hbm.at[pl.ds(off, CHUNK)])


@functools.cache
def _sc_gather():
    return functools.partial(
        pl.kernel,
        out_type=jax.ShapeDtypeStruct((E, IN), jnp.float32),
        mesh=_sc_mesh(),
        scratch_types=[
            pltpu.VMEM((CHUNK,), jnp.int32),
            pltpu.VMEM((CHUNK, IN), jnp.float32),
            pltpu.SemaphoreType.DMA,
        ],
        compiler_params=pltpu.CompilerParams(use_tc_tiling_on_sc=False),
    )(_gather_body)


ROWS_PER_TILE = N // NS


def _scatter_body(msg_hbm, dst_hbm, zero_hbm, out_hbm, idx_v, msg_v, agg_sh):
    cid = lax.axis_index("c")
    sid = lax.axis_index("s")
    wid = sid * NC + cid
    r0 = sid * ROWS_PER_TILE
    pltpu.sync_copy(zero_hbm.at[pl.ds(r0, ROWS_PER_TILE)],
                    agg_sh.at[pl.ds(r0, ROWS_PER_TILE)])
    plsc.subcore_barrier()
    base = wid * PER_W
    for c in range(NCHUNK):
        off = base + c * CHUNK
        pltpu.sync_copy(dst_hbm.at[pl.ds(off, CHUNK)], idx_v)
        pltpu.sync_copy(msg_hbm.at[pl.ds(off, CHUNK)], msg_v)
        pltpu.sync_copy(msg_v, agg_sh.at[idx_v], add=True)
    plsc.subcore_barrier()
    pltpu.sync_copy(agg_sh.at[pl.ds(r0, ROWS_PER_TILE)],
                    out_hbm.at[cid, pl.ds(r0, ROWS_PER_TILE)])


@functools.cache
def _sc_scatter():
    return functools.partial(
        pl.kernel,
        out_type=jax.ShapeDtypeStruct((NC, N, HID), jnp.float32),
        mesh=_sc_mesh(),
        scratch_types=[
            pltpu.VMEM((CHUNK,), jnp.int32),
            pltpu.VMEM((CHUNK, HID), jnp.float32),
            pltpu.VMEM_SHARED((N, HID), jnp.float32),
        ],
        compiler_params=pltpu.CompilerParams(use_tc_tiling_on_sc=False),
    )(_scatter_body)


TE = 2000


def _msg_body(xj_ref, ea_ref, a2_ref, b_ref, out_ref):
    xj = xj_ref[...]
    p = jnp.dot(xj, a2_ref[...], preferred_element_type=jnp.float32)
    acc = jnp.dot(xj, b_ref[...], preferred_element_type=jnp.float32)
    ea = ea_ref[...]
    for k in range(EDGE):
        acc = acc + ea[:, k:k + 1] * p[:, k * HID:(k + 1) * HID]
    out_ref[...] = acc


def _msg_kernel(x_j, ea, a2, bmat):
    return pl.pallas_call(
        _msg_body,
        grid=(E // TE,),
        in_specs=[
            pl.BlockSpec((TE, IN), lambda i: (i, 0)),
            pl.BlockSpec((TE, EDGE), lambda i: (i, 0)),
            pl.BlockSpec((IN, EDGE * HID), lambda i: (0, 0)),
            pl.BlockSpec((IN, HID), lambda i: (0, 0)),
        ],
        out_specs=pl.BlockSpec((TE, HID), lambda i: (i, 0)),
        out_shape=jax.ShapeDtypeStruct((E, HID), jnp.float32),
        compiler_params=pltpu.CompilerParams(
            dimension_semantics=("arbitrary",)),
    )(x_j, ea, a2, bmat)


def _tail_body(agg_ref, x_ref, root_ref, cb_ref, g_ref, be_ref,
               sw_ref, sb_ref, tw_ref, tb_ref, batch_ref, out_ref):
    h = (agg_ref[0] + agg_ref[1]
         + jnp.dot(x_ref[...], root_ref[...], preferred_element_type=jnp.float32)
         + cb_ref[...])
    mean = jnp.mean(h, axis=0, keepdims=True)
    var = jnp.mean((h - mean) * (h - mean), axis=0, keepdims=True)
    hn = (h - mean) * lax.rsqrt(var + 1e-5) * g_ref[...] + be_ref[...]
    hr = jnp.maximum(hn, 0.0)
    s_x = jnp.dot(hr, sw_ref[...], preferred_element_type=jnp.float32) + sb_ref[...]
    t_x = jnp.dot(hr, tw_ref[...], preferred_element_type=jnp.float32) + tb_ref[...]
    s_x = jnp.clip(s_x, -30.0, 30.0)
    f_x = 1.0 / (1.0 + jnp.exp(s_x)) * jnp.tanh(t_x)
    gids = lax.broadcasted_iota(jnp.int32, (G, N), 0)
    onehot = jnp.where(gids == batch_ref[...], 1.0, 0.0)
    out_ref[...] = jnp.dot(onehot, f_x, preferred_element_type=jnp.float32)


def _tail_kernel(agg, x, root, conv_bias, bn_gamma, bn_beta,
                 s_wt, s_b, t_wt, t_b, batchf):
    return pl.pallas_call(
        _tail_body,
        out_shape=jax.ShapeDtypeStruct((G, OUT), jnp.float32),
    )(agg, x, root, conv_bias, bn_gamma, bn_beta,
      s_wt, s_b, t_wt, t_b, batchf)


def kernel(x, edge_index, edge_attr, batch, edge_batch, nn_W, nn_b, root,
           conv_bias, bn_gamma, bn_beta, s_W, s_b, t_W, t_b):
    src = edge_index[0].astype(jnp.int32)
    dst = edge_index[1].astype(jnp.int32)
    a2 = nn_W.reshape(IN, HID, EDGE).transpose(0, 2, 1).reshape(IN, EDGE * HID)
    bmat = nn_b.reshape(IN, HID)

    x_j = _sc_gather()(x, src)
    msg = _msg_kernel(x_j, edge_attr, a2, bmat)
    zeros = jnp.zeros((N, HID), jnp.float32)
    agg = _sc_scatter()(msg, dst, zeros)

    feature = _tail_kernel(
        agg, x, root,
        conv_bias.reshape(1, HID), bn_gamma.reshape(1, HID),
        bn_beta.reshape(1, HID),
        s_W.T, s_b.reshape(1, OUT), t_W.T, t_b.reshape(1, OUT),
        batch.astype(jnp.int32).reshape(1, N))
    return feature


# speedup vs baseline: 3.1205x; 1.3152x over previous
import functools

import jax
import jax.numpy as jnp
from jax import lax
from jax.experimental import pallas as pl
from jax.experimental.pallas import tpu as pltpu
from jax.experimental.pallas import tpu_sc as plsc

N = 10000
E = 160000
IN = 64
HID = 32
EDGE = 16
OUT = 128
G = 64

YW = EDGE * HID + HID

NC = 2
NS = 16
NW = NC * NS
PER_W = E // NW
CHUNK = 40
NCHUNK = PER_W // CHUNK
ROWS_PER_TILE = N // NS


def _sc_mesh():
    return plsc.VectorSubcoreMesh(core_axis_name="c", subcore_axis_name="s")


def _edge_body(y_hbm, src_hbm, dst_hbm, ea_hbm, zero_hbm, out_hbm,
               idx_v, dst_v, ea_v, rows_v, msg_v, agg_sh, sem):
    cid = lax.axis_index("c")
    sid = lax.axis_index("s")
    wid = sid * NC + cid
    r0 = sid * ROWS_PER_TILE
    pltpu.sync_copy(zero_hbm.at[pl.ds(r0, ROWS_PER_TILE)],
                    agg_sh.at[pl.ds(r0, ROWS_PER_TILE)])
    plsc.subcore_barrier()
    base = wid * PER_W

    def chunk_body(c, carry):
        off = pl.multiple_of(base + c * CHUNK, 8)
        off16 = pl.multiple_of((base + c * CHUNK) * EDGE, 8)
        pltpu.sync_copy(src_hbm.at[pl.ds(off, CHUNK)], idx_v)
        pltpu.sync_copy(dst_hbm.at[pl.ds(off, CHUNK)], dst_v)
        pltpu.sync_copy(ea_hbm.at[pl.ds(off16, CHUNK * EDGE)], ea_v)
        pltpu.async_copy(y_hbm.at[idx_v], rows_v, sem).wait()

        def edge_body(e, carry2):
            ea_row = ea_v[pl.ds(e * EDGE, EDGE)]
            acc0 = rows_v[e, pl.ds(EDGE * HID, 16)]
            acc1 = rows_v[e, pl.ds(EDGE * HID + 16, 16)]
            for k in range(EDGE):
                s = ea_row[k]
                acc0 = acc0 + s * rows_v[e, pl.ds(k * HID, 16)]
                acc1 = acc1 + s * rows_v[e, pl.ds(k * HID + 16, 16)]
            msg_v[e, pl.ds(0, 16)] = acc0
            msg_v[e, pl.ds(16, 16)] = acc1
            return carry2

        lax.fori_loop(0, CHUNK, edge_body, 0)
        pltpu.sync_copy(msg_v, agg_sh.at[dst_v], add=True)
        return carry

    lax.fori_loop(0, NCHUNK, chunk_body, 0)
    plsc.subcore_barrier()
    pltpu.sync_copy(agg_sh.at[pl.ds(r0, ROWS_PER_TILE)],
                    out_hbm.at[cid, pl.ds(r0, ROWS_PER_TILE)])


@functools.cache
def _sc_edge():
    return functools.partial(
        pl.kernel,
        out_type=jax.ShapeDtypeStruct((NC, N, HID), jnp.float32),
        mesh=_sc_mesh(),
        scratch_types=[
            pltpu.VMEM((CHUNK,), jnp.int32),
            pltpu.VMEM((CHUNK,), jnp.int32),
            pltpu.VMEM((CHUNK * EDGE,), jnp.float32),
            pltpu.VMEM((CHUNK, YW), jnp.float32),
            pltpu.VMEM((CHUNK, HID), jnp.float32),
            pltpu.VMEM_SHARED((N, HID), jnp.float32),
            pltpu.SemaphoreType.DMA,
        ],
        compiler_params=pltpu.CompilerParams(use_tc_tiling_on_sc=False),
    )(_edge_body)


TN = 2000


def _y_body(x_ref, a_ref, out_ref):
    out_ref[...] = jnp.dot(x_ref[...], a_ref[...],
                           preferred_element_type=jnp.float32)


def _y_kernel(x, a2ext):
    return pl.pallas_call(
        _y_body,
        grid=(N // TN,),
        in_specs=[
            pl.BlockSpec((TN, IN), lambda i: (i, 0)),
            pl.BlockSpec((IN, YW), lambda i: (0, 0)),
        ],
        out_specs=pl.BlockSpec((TN, YW), lambda i: (i, 0)),
        out_shape=jax.ShapeDtypeStruct((N, YW), jnp.float32),
        compiler_params=pltpu.CompilerParams(
            dimension_semantics=("arbitrary",)),
    )(x, a2ext)


def _tail_body(agg_ref, x_ref, root_ref, cb_ref, g_ref, be_ref,
               sw_ref, sb_ref, tw_ref, tb_ref, batch_ref, out_ref):
    h = (agg_ref[0] + agg_ref[1]
         + jnp.dot(x_ref[...], root_ref[...], preferred_element_type=jnp.float32)
         + cb_ref[...])
    mean = jnp.mean(h, axis=0, keepdims=True)
    var = jnp.mean((h - mean) * (h - mean), axis=0, keepdims=True)
    hn = (h - mean) * lax.rsqrt(var + 1e-5) * g_ref[...] + be_ref[...]
    hr = jnp.maximum(hn, 0.0)
    s_x = jnp.dot(hr, sw_ref[...], preferred_element_type=jnp.float32) + sb_ref[...]
    t_x = jnp.dot(hr, tw_ref[...], preferred_element_type=jnp.float32) + tb_ref[...]
    s_x = jnp.clip(s_x, -30.0, 30.0)
    f_x = 1.0 / (1.0 + jnp.exp(s_x)) * jnp.tanh(t_x)
    gids = lax.broadcasted_iota(jnp.int32, (G, N), 0)
    onehot = jnp.where(gids == batch_ref[...], 1.0, 0.0)
    out_ref[...] = jnp.dot(onehot, f_x, preferred_element_type=jnp.float32)


def _tail_kernel(agg, x, root, conv_bias, bn_gamma, bn_beta,
                 s_wt, s_b, t_wt, t_b, batchi):
    return pl.pallas_call(
        _tail_body,
        out_shape=jax.ShapeDtypeStruct((G, OUT), jnp.float32),
    )(agg, x, root, conv_bias, bn_gamma, bn_beta,
      s_wt, s_b, t_wt, t_b, batchi)


def kernel(x, edge_index, edge_attr, batch, edge_batch, nn_W, nn_b, root,
           conv_bias, bn_gamma, bn_beta, s_W, s_b, t_W, t_b):
    src = edge_index[0].astype(jnp.int32)
    dst = edge_index[1].astype(jnp.int32)
    a2 = nn_W.reshape(IN, HID, EDGE).transpose(0, 2, 1).reshape(IN, EDGE * HID)
    a2ext = jnp.concatenate([a2, nn_b.reshape(IN, HID)], axis=1)

    y = _y_kernel(x, a2ext)
    zeros = jnp.zeros((N, HID), jnp.float32)
    agg = _sc_edge()(y, src, dst, edge_attr.reshape(-1), zeros)

    feature = _tail_kernel(
        agg, x, root,
        conv_bias.reshape(1, HID), bn_gamma.reshape(1, HID),
        bn_beta.reshape(1, HID),
        s_W.T, s_b.reshape(1, OUT), t_W.T, t_b.reshape(1, OUT),
        batch.astype(jnp.int32).reshape(1, N))
    return feature


# speedup vs baseline: 5.8463x; 1.8735x over previous
import functools

import jax
import jax.numpy as jnp
from jax import lax
from jax.experimental import pallas as pl
from jax.experimental.pallas import tpu as pltpu
from jax.experimental.pallas import tpu_sc as plsc

N = 10000
E = 160000
IN = 64
HID = 32
EDGE = 16
OUT = 128
G = 64

YW = EDGE * HID + HID

NC = 2
NS = 16
NW = NC * NS
PER_W = E // NW
GROUP = 1000
NGROUP = PER_W // GROUP
SUB = 40
NSUB = GROUP // SUB
ROWS_PER_TILE = N // NS


def _sc_mesh():
    return plsc.VectorSubcoreMesh(core_axis_name="c", subcore_axis_name="s")


def _edge_body(y_hbm, src_hbm, dst_hbm, ea_hbm, zero_hbm, out_hbm,
               src_g, dst_g, ea_g, rows0, rows1, msg_g, agg_sh, sem0, sem1):
    cid = lax.axis_index("c")
    sid = lax.axis_index("s")
    wid = sid * NC + cid
    r0 = sid * ROWS_PER_TILE
    pltpu.sync_copy(zero_hbm.at[pl.ds(r0, ROWS_PER_TILE)],
                    agg_sh.at[pl.ds(r0, ROWS_PER_TILE)])
    plsc.subcore_barrier()
    base = wid * PER_W

    def compute_sub(rows_b, s):
        def edge_body(e, carry2):
            eg = s * SUB + e
            ea_row = ea_g[pl.ds(eg * EDGE, EDGE)]
            acc0 = rows_b[e, pl.ds(EDGE * HID, 16)]
            acc1 = rows_b[e, pl.ds(EDGE * HID + 16, 16)]
            for k in range(EDGE):
                w = ea_row[k]
                acc0 = acc0 + w * rows_b[e, pl.ds(k * HID, 16)]
                acc1 = acc1 + w * rows_b[e, pl.ds(k * HID + 16, 16)]
            msg_g[eg, pl.ds(0, 16)] = acc0
            msg_g[eg, pl.ds(16, 16)] = acc1
            return carry2

        lax.fori_loop(0, SUB, edge_body, 0)

    def gather_start(s, rows_b, sem_b):
        pltpu.async_copy(y_hbm.at[src_g.at[pl.ds(s * SUB, SUB)]],
                         rows_b, sem_b)

    def gather_wait(s, rows_b, sem_b):
        pltpu.make_async_copy(y_hbm.at[src_g.at[pl.ds(s * SUB, SUB)]],
                              rows_b, sem_b).wait()

    for g in range(NGROUP):
        goff = pl.multiple_of(base + g * GROUP, 8)
        goff16 = pl.multiple_of((base + g * GROUP) * EDGE, 8)
        pltpu.sync_copy(src_hbm.at[pl.ds(goff, GROUP)], src_g)
        pltpu.sync_copy(dst_hbm.at[pl.ds(goff, GROUP)], dst_g)
        pltpu.sync_copy(ea_hbm.at[pl.ds(goff16, GROUP * EDGE)], ea_g)
        gather_start(0, rows0, sem0)

        def pair_body(i, carry):
            s0 = 2 * i
            s1 = 2 * i + 1
            gather_start(s1, rows1, sem1)
            gather_wait(s0, rows0, sem0)
            compute_sub(rows0, s0)
            gather_start(s0 + 2, rows0, sem0)
            gather_wait(s1, rows1, sem1)
            compute_sub(rows1, s1)
            return carry

        lax.fori_loop(0, NSUB // 2, pair_body, 0)
        gather_wait(NSUB - 1, rows0, sem0)
        compute_sub(rows0, NSUB - 1)
        pltpu.sync_copy(msg_g, agg_sh.at[dst_g], add=True)

    plsc.subcore_barrier()
    pltpu.sync_copy(agg_sh.at[pl.ds(r0, ROWS_PER_TILE)],
                    out_hbm.at[cid, pl.ds(r0, ROWS_PER_TILE)])


@functools.cache
def _sc_edge():
    return functools.partial(
        pl.kernel,
        out_type=jax.ShapeDtypeStruct((NC, N, HID), jnp.float32),
        mesh=_sc_mesh(),
        scratch_types=[
            pltpu.VMEM((GROUP,), jnp.int32),
            pltpu.VMEM((GROUP,), jnp.int32),
            pltpu.VMEM((GROUP * EDGE,), jnp.float32),
            pltpu.VMEM((SUB, YW), jnp.float32),
            pltpu.VMEM((SUB, YW), jnp.float32),
            pltpu.VMEM((GROUP, HID), jnp.float32),
            pltpu.VMEM_SHARED((N, HID), jnp.float32),
            pltpu.SemaphoreType.DMA,
            pltpu.SemaphoreType.DMA,
        ],
        compiler_params=pltpu.CompilerParams(use_tc_tiling_on_sc=False),
    )(_edge_body)


TN = 2000


def _y_body(x_ref, a_ref, out_ref):
    out_ref[...] = jnp.dot(x_ref[...], a_ref[...],
                           preferred_element_type=jnp.float32)


def _y_kernel(x, a2ext):
    return pl.pallas_call(
        _y_body,
        grid=(N // TN,),
        in_specs=[
            pl.BlockSpec((TN, IN), lambda i: (i, 0)),
            pl.BlockSpec((IN, YW), lambda i: (0, 0)),
        ],
        out_specs=pl.BlockSpec((TN, YW), lambda i: (i, 0)),
        out_shape=jax.ShapeDtypeStruct((N, YW), jnp.float32),
        compiler_params=pltpu.CompilerParams(
            dimension_semantics=("arbitrary",)),
    )(x, a2ext)


def _tail_body(agg_ref, x_ref, root_ref, cb_ref, g_ref, be_ref,
               sw_ref, sb_ref, tw_ref, tb_ref, batch_ref, out_ref):
    h = (agg_ref[0] + agg_ref[1]
         + jnp.dot(x_ref[...], root_ref[...], preferred_element_type=jnp.float32)
         + cb_ref[...])
    mean = jnp.mean(h, axis=0, keepdims=True)
    var = jnp.mean((h - mean) * (h - mean), axis=0, keepdims=True)
    hn = (h - mean) * lax.rsqrt(var + 1e-5) * g_ref[...] + be_ref[...]
    hr = jnp.maximum(hn, 0.0)
    s_x = jnp.dot(hr, sw_ref[...], preferred_element_type=jnp.float32) + sb_ref[...]
    t_x = jnp.dot(hr, tw_ref[...], preferred_element_type=jnp.float32) + tb_ref[...]
    s_x = jnp.clip(s_x, -30.0, 30.0)
    f_x = 1.0 / (1.0 + jnp.exp(s_x)) * jnp.tanh(t_x)
    gids = lax.broadcasted_iota(jnp.int32, (G, N), 0)
    onehot = jnp.where(gids == batch_ref[...], 1.0, 0.0)
    out_ref[...] = jnp.dot(onehot, f_x, preferred_element_type=jnp.float32)


def _tail_kernel(agg, x, root, conv_bias, bn_gamma, bn_beta,
                 s_wt, s_b, t_wt, t_b, batchi):
    return pl.pallas_call(
        _tail_body,
        out_shape=jax.ShapeDtypeStruct((G, OUT), jnp.float32),
    )(agg, x, root, conv_bias, bn_gamma, bn_beta,
      s_wt, s_b, t_wt, t_b, batchi)


def kernel(x, edge_index, edge_attr, batch, edge_batch, nn_W, nn_b, root,
           conv_bias, bn_gamma, bn_beta, s_W, s_b, t_W, t_b):
    src = edge_index[0].astype(jnp.int32)
    dst = edge_index[1].astype(jnp.int32)
    a2 = nn_W.reshape(IN, HID, EDGE).transpose(0, 2, 1).reshape(IN, EDGE * HID)
    a2ext = jnp.concatenate([a2, nn_b.reshape(IN, HID)], axis=1)

    y = _y_kernel(x, a2ext)
    zeros = jnp.zeros((N, HID), jnp.float32)
    agg = _sc_edge()(y, src, dst, edge_attr.reshape(-1), zeros)

    feature = _tail_kernel(
        agg, x, root,
        conv_bias.reshape(1, HID), bn_gamma.reshape(1, HID),
        bn_beta.reshape(1, HID),
        s_W.T, s_b.reshape(1, OUT), t_W.T, t_b.reshape(1, OUT),
        batch.astype(jnp.int32).reshape(1, N))
    return feature


# speedup vs baseline: 5.9199x; 1.0126x over previous
import functools

import jax
import jax.numpy as jnp
import numpy as np
from jax import lax
from jax.experimental import pallas as pl
from jax.experimental.pallas import tpu as pltpu
from jax.experimental.pallas import tpu_sc as plsc

N = 10000
E = 160000
IN = 64
HID = 32
EDGE = 16
OUT = 128
G = 64

YW = EDGE * HID + HID

NC = 2
NS = 16
NW = NC * NS
PER_W = E // NW
GROUP = 1000
NGROUP = PER_W // GROUP
SUB = 40
NSUB = GROUP // SUB
ROWS_PER_TILE = N // NS


def _sc_mesh():
    return plsc.VectorSubcoreMesh(core_axis_name="c", subcore_axis_name="s")


def _edge_body(y_hbm, src_hbm, dst_hbm, ea_hbm, zero_hbm, out_hbm,
               src_g, dst_g, ea_g, rows0, rows1, msg_g, agg_sh, sem0, sem1):
    cid = lax.axis_index("c")
    sid = lax.axis_index("s")
    wid = sid * NC + cid
    r0 = sid * ROWS_PER_TILE
    pltpu.sync_copy(zero_hbm.at[pl.ds(r0, ROWS_PER_TILE)],
                    agg_sh.at[pl.ds(r0, ROWS_PER_TILE)])
    plsc.subcore_barrier()
    base = wid * PER_W

    def compute_sub(rows_b, s):
        def edge_body(e, carry2):
            eg = s * SUB + e
            ea_row = ea_g[pl.ds(eg * EDGE, EDGE)]
            acc0, acc1 = plsc.unpack(rows_b[e, pl.ds(EDGE * HID, 32)],
                                     format=plsc.PackFormat.INTERLEAVED)
            for k in range(EDGE):
                w = ea_row[k]
                y0, y1 = plsc.unpack(rows_b[e, pl.ds(k * HID, 32)],
                                     format=plsc.PackFormat.INTERLEAVED)
                acc0 = acc0 + w * y0
                acc1 = acc1 + w * y1
            msg_g[eg, pl.ds(0, 16)] = acc0
            msg_g[eg, pl.ds(16, 16)] = acc1
            return carry2

        lax.fori_loop(0, SUB, edge_body, 0)

    def gather_start(s, rows_b, sem_b):
        pltpu.async_copy(y_hbm.at[src_g.at[pl.ds(s * SUB, SUB)]],
                         rows_b, sem_b)

    def gather_wait(s, rows_b, sem_b):
        pltpu.make_async_copy(y_hbm.at[src_g.at[pl.ds(s * SUB, SUB)]],
                              rows_b, sem_b).wait()

    for g in range(NGROUP):
        goff = pl.multiple_of(base + g * GROUP, 8)
        goff16 = pl.multiple_of((base + g * GROUP) * EDGE, 8)
        pltpu.sync_copy(src_hbm.at[pl.ds(goff, GROUP)], src_g)
        pltpu.sync_copy(dst_hbm.at[pl.ds(goff, GROUP)], dst_g)
        pltpu.sync_copy(ea_hbm.at[pl.ds(goff16, GROUP * EDGE)], ea_g)
        gather_start(0, rows0, sem0)

        def pair_body(i, carry):
            s0 = 2 * i
            s1 = 2 * i + 1
            gather_start(s1, rows1, sem1)
            gather_wait(s0, rows0, sem0)
            compute_sub(rows0, s0)
            gather_start(s0 + 2, rows0, sem0)
            gather_wait(s1, rows1, sem1)
            compute_sub(rows1, s1)
            return carry

        lax.fori_loop(0, NSUB // 2, pair_body, 0)
        gather_wait(NSUB - 1, rows0, sem0)
        compute_sub(rows0, NSUB - 1)
        pltpu.sync_copy(msg_g, agg_sh.at[dst_g], add=True)

    plsc.subcore_barrier()
    pltpu.sync_copy(agg_sh.at[pl.ds(r0, ROWS_PER_TILE)],
                    out_hbm.at[cid, pl.ds(r0, ROWS_PER_TILE)])


@functools.cache
def _sc_edge():
    return functools.partial(
        pl.kernel,
        out_type=jax.ShapeDtypeStruct((NC, N, HID), jnp.float32),
        mesh=_sc_mesh(),
        scratch_types=[
            pltpu.VMEM((GROUP,), jnp.int32),
            pltpu.VMEM((GROUP,), jnp.int32),
            pltpu.VMEM((GROUP * EDGE,), jnp.float32),
            pltpu.VMEM((SUB, YW), jnp.bfloat16),
            pltpu.VMEM((SUB, YW), jnp.bfloat16),
            pltpu.VMEM((GROUP, HID), jnp.float32),
            pltpu.VMEM_SHARED((N, HID), jnp.float32),
            pltpu.SemaphoreType.DMA,
            pltpu.SemaphoreType.DMA,
        ],
        compiler_params=pltpu.CompilerParams(use_tc_tiling_on_sc=False,
                                             needs_layout_passes=False),
    )(_edge_body)


TN = 2000


def _y_body(x_ref, a_ref, out_ref):
    out_ref[...] = jnp.dot(x_ref[...], a_ref[...],
                           preferred_element_type=jnp.float32
                           ).astype(jnp.bfloat16)


def _y_kernel(x, a2ext):
    return pl.pallas_call(
        _y_body,
        grid=(N // TN,),
        in_specs=[
            pl.BlockSpec((TN, IN), lambda i: (i, 0)),
            pl.BlockSpec((IN, YW), lambda i: (0, 0)),
        ],
        out_specs=pl.BlockSpec((TN, YW), lambda i: (i, 0)),
        out_shape=jax.ShapeDtypeStruct((N, YW), jnp.bfloat16),
        compiler_params=pltpu.CompilerParams(
            dimension_semantics=("arbitrary",)),
    )(x, a2ext)


def _tail_body(agg_ref, x_ref, root_ref, cb_ref, g_ref, be_ref,
               sw_ref, sb_ref, tw_ref, tb_ref, batch_ref, out_ref):
    h = (agg_ref[0] + agg_ref[1]
         + jnp.dot(x_ref[...], root_ref[...], preferred_element_type=jnp.float32)
         + cb_ref[...])
    mean = jnp.mean(h, axis=0, keepdims=True)
    var = jnp.mean((h - mean) * (h - mean), axis=0, keepdims=True)
    hn = (h - mean) * lax.rsqrt(var + 1e-5) * g_ref[...] + be_ref[...]
    hr = jnp.maximum(hn, 0.0)
    s_x = jnp.dot(hr, sw_ref[...], preferred_element_type=jnp.float32) + sb_ref[...]
    t_x = jnp.dot(hr, tw_ref[...], preferred_element_type=jnp.float32) + tb_ref[...]
    s_x = jnp.clip(s_x, -30.0, 30.0)
    f_x = 1.0 / (1.0 + jnp.exp(s_x)) * jnp.tanh(t_x)
    gids = lax.broadcasted_iota(jnp.int32, (G, N), 0)
    onehot = jnp.where(gids == batch_ref[...], 1.0, 0.0)
    out_ref[...] = jnp.dot(onehot, f_x, preferred_element_type=jnp.float32)


def _tail_kernel(agg, x, root, conv_bias, bn_gamma, bn_beta,
                 s_wt, s_b, t_wt, t_b, batchi):
    return pl.pallas_call(
        _tail_body,
        out_shape=jax.ShapeDtypeStruct((G, OUT), jnp.float32),
    )(agg, x, root, conv_bias, bn_gamma, bn_beta,
      s_wt, s_b, t_wt, t_b, batchi)


def kernel(x, edge_index, edge_attr, batch, edge_batch, nn_W, nn_b, root,
           conv_bias, bn_gamma, bn_beta, s_W, s_b, t_W, t_b):
    src = edge_index[0].astype(jnp.int32)
    dst = edge_index[1].astype(jnp.int32)
    a2 = nn_W.reshape(IN, HID, EDGE).transpose(0, 2, 1).reshape(IN, EDGE * HID)
    a2ext = jnp.concatenate([a2, nn_b.reshape(IN, HID)], axis=1)
    blk = np.stack([np.arange(16), np.arange(16) + 16], axis=1).reshape(-1)
    perm = (np.arange(YW // HID)[:, None] * HID + blk[None, :]).reshape(-1)
    a2ext = a2ext[:, perm]

    y = _y_kernel(x, a2ext)
    zeros = jnp.zeros((N, HID), jnp.float32)
    agg = _sc_edge()(y, src, dst, edge_attr.reshape(-1), zeros)

    feature = _tail_kernel(
        agg, x, root,
        conv_bias.reshape(1, HID), bn_gamma.reshape(1, HID),
        bn_beta.reshape(1, HID),
        s_W.T, s_b.reshape(1, OUT), t_W.T, t_b.reshape(1, OUT),
        batch.astype(jnp.int32).reshape(1, N))
    return feature


# speedup vs baseline: 6.1452x; 1.0381x over previous
import functools

import jax
import jax.numpy as jnp
import numpy as np
from jax import lax
from jax.experimental import pallas as pl
from jax.experimental.pallas import tpu as pltpu
from jax.experimental.pallas import tpu_sc as plsc

N = 10000
E = 160000
IN = 64
HID = 32
EDGE = 16
OUT = 128
G = 64

YW = EDGE * HID + HID

NC = 2
NS = 16
NW = NC * NS
PER_W = E // NW
GROUP = 1000
NGROUP = PER_W // GROUP
SUB = 40
NSUB = GROUP // SUB
ROWS_PER_TILE = N // NS


def _sc_mesh():
    return plsc.VectorSubcoreMesh(core_axis_name="c", subcore_axis_name="s")


def _edge_body(y_hbm, src_hbm, dst_hbm, ea_hbm, zero_hbm, out_hbm,
               src_g, dst_g, ea_g, rows0, rows1, msg_g, agg_sh, sem0, sem1):
    cid = lax.axis_index("c")
    sid = lax.axis_index("s")
    wid = sid * NC + cid
    r0 = sid * ROWS_PER_TILE
    pltpu.sync_copy(zero_hbm.at[pl.ds(r0, ROWS_PER_TILE)],
                    agg_sh.at[pl.ds(r0, ROWS_PER_TILE)])
    plsc.subcore_barrier()
    base = wid * PER_W

    def compute_sub(rows_b, s):
        @plsc.parallel_loop(0, SUB, unroll=4)
        def edge_body(e):
            eg = s * SUB + e
            ea_row = ea_g[pl.ds(eg * EDGE, EDGE)]
            acc0, acc1 = plsc.unpack(rows_b[e, pl.ds(EDGE * HID, 32)],
                                     format=plsc.PackFormat.INTERLEAVED)
            for k in range(EDGE):
                w = ea_row[k]
                y0, y1 = plsc.unpack(rows_b[e, pl.ds(k * HID, 32)],
                                     format=plsc.PackFormat.INTERLEAVED)
                acc0 = acc0 + w * y0
                acc1 = acc1 + w * y1
            msg_g[eg, pl.ds(0, 16)] = acc0
            msg_g[eg, pl.ds(16, 16)] = acc1

    def gather_start(s, rows_b, sem_b):
        pltpu.async_copy(y_hbm.at[src_g.at[pl.ds(s * SUB, SUB)]],
                         rows_b, sem_b)

    def gather_wait(s, rows_b, sem_b):
        pltpu.make_async_copy(y_hbm.at[src_g.at[pl.ds(s * SUB, SUB)]],
                              rows_b, sem_b).wait()

    for g in range(NGROUP):
        goff = pl.multiple_of(base + g * GROUP, 8)
        goff16 = pl.multiple_of((base + g * GROUP) * EDGE, 8)
        pltpu.sync_copy(src_hbm.at[pl.ds(goff, GROUP)], src_g)
        pltpu.sync_copy(dst_hbm.at[pl.ds(goff, GROUP)], dst_g)
        pltpu.sync_copy(ea_hbm.at[pl.ds(goff16, GROUP * EDGE)], ea_g)
        gather_start(0, rows0, sem0)

        def pair_body(i, carry):
            s0 = 2 * i
            s1 = 2 * i + 1
            gather_start(s1, rows1, sem1)
            gather_wait(s0, rows0, sem0)
            compute_sub(rows0, s0)
            gather_start(s0 + 2, rows0, sem0)
            gather_wait(s1, rows1, sem1)
            compute_sub(rows1, s1)
            return carry

        lax.fori_loop(0, NSUB // 2, pair_body, 0)
        gather_wait(NSUB - 1, rows0, sem0)
        compute_sub(rows0, NSUB - 1)
        pltpu.sync_copy(msg_g, agg_sh.at[dst_g], add=True)

    plsc.subcore_barrier()
    pltpu.sync_copy(agg_sh.at[pl.ds(r0, ROWS_PER_TILE)],
                    out_hbm.at[cid, pl.ds(r0, ROWS_PER_TILE)])


@functools.cache
def _sc_edge():
    return functools.partial(
        pl.kernel,
        out_type=jax.ShapeDtypeStruct((NC, N, HID), jnp.float32),
        mesh=_sc_mesh(),
        scratch_types=[
            pltpu.VMEM((GROUP,), jnp.int32),
            pltpu.VMEM((GROUP,), jnp.int32),
            pltpu.VMEM((GROUP * EDGE,), jnp.float32),
            pltpu.VMEM((SUB, YW), jnp.bfloat16),
            pltpu.VMEM((SUB, YW), jnp.bfloat16),
            pltpu.VMEM((GROUP, HID), jnp.float32),
            pltpu.VMEM_SHARED((N, HID), jnp.float32),
            pltpu.SemaphoreType.DMA,
            pltpu.SemaphoreType.DMA,
        ],
        compiler_params=pltpu.CompilerParams(use_tc_tiling_on_sc=False,
                                             needs_layout_passes=False),
    )(_edge_body)


TN = 2000


def _y_body(x_ref, a_ref, out_ref):
    out_ref[...] = jnp.dot(x_ref[...], a_ref[...],
                           preferred_element_type=jnp.float32
                           ).astype(jnp.bfloat16)


def _y_kernel(x, a2ext):
    return pl.pallas_call(
        _y_body,
        grid=(N // TN,),
        in_specs=[
            pl.BlockSpec((TN, IN), lambda i: (i, 0)),
            pl.BlockSpec((IN, YW), lambda i: (0, 0)),
        ],
        out_specs=pl.BlockSpec((TN, YW), lambda i: (i, 0)),
        out_shape=jax.ShapeDtypeStruct((N, YW), jnp.bfloat16),
        compiler_params=pltpu.CompilerParams(
            dimension_semantics=("arbitrary",)),
    )(x, a2ext)


def _tail_body(agg_ref, x_ref, root_ref, cb_ref, g_ref, be_ref,
               sw_ref, sb_ref, tw_ref, tb_ref, batch_ref, out_ref):
    h = (agg_ref[0] + agg_ref[1]
         + jnp.dot(x_ref[...], root_ref[...], preferred_element_type=jnp.float32)
         + cb_ref[...])
    mean = jnp.mean(h, axis=0, keepdims=True)
    var = jnp.mean((h - mean) * (h - mean), axis=0, keepdims=True)
    hn = (h - mean) * lax.rsqrt(var + 1e-5) * g_ref[...] + be_ref[...]
    hr = jnp.maximum(hn, 0.0)
    s_x = jnp.dot(hr, sw_ref[...], preferred_element_type=jnp.float32) + sb_ref[...]
    t_x = jnp.dot(hr, tw_ref[...], preferred_element_type=jnp.float32) + tb_ref[...]
    s_x = jnp.clip(s_x, -30.0, 30.0)
    f_x = 1.0 / (1.0 + jnp.exp(s_x)) * jnp.tanh(t_x)
    gids = lax.broadcasted_iota(jnp.int32, (G, N), 0)
    onehot = jnp.where(gids == batch_ref[...], 1.0, 0.0)
    out_ref[...] = jnp.dot(onehot, f_x, preferred_element_type=jnp.float32)


def _tail_kernel(agg, x, root, conv_bias, bn_gamma, bn_beta,
                 s_wt, s_b, t_wt, t_b, batchi):
    return pl.pallas_call(
        _tail_body,
        out_shape=jax.ShapeDtypeStruct((G, OUT), jnp.float32),
    )(agg, x, root, conv_bias, bn_gamma, bn_beta,
      s_wt, s_b, t_wt, t_b, batchi)


def kernel(x, edge_index, edge_attr, batch, edge_batch, nn_W, nn_b, root,
           conv_bias, bn_gamma, bn_beta, s_W, s_b, t_W, t_b):
    src = edge_index[0].astype(jnp.int32)
    dst = edge_index[1].astype(jnp.int32)
    a2 = nn_W.reshape(IN, HID, EDGE).transpose(0, 2, 1).reshape(IN, EDGE * HID)
    a2ext = jnp.concatenate([a2, nn_b.reshape(IN, HID)], axis=1)
    blk = np.stack([np.arange(16), np.arange(16) + 16], axis=1).reshape(-1)
    perm = (np.arange(YW // HID)[:, None] * HID + blk[None, :]).reshape(-1)
    a2ext = a2ext[:, perm]

    y = _y_kernel(x, a2ext)
    zeros = jnp.zeros((N, HID), jnp.float32)
    agg = _sc_edge()(y, src, dst, edge_attr.reshape(-1), zeros)

    feature = _tail_kernel(
        agg, x, root,
        conv_bias.reshape(1, HID), bn_gamma.reshape(1, HID),
        bn_beta.reshape(1, HID),
        s_W.T, s_b.reshape(1, OUT), t_W.T, t_b.reshape(1, OUT),
        batch.astype(jnp.int32).reshape(1, N))
    return feature


# speedup vs baseline: 6.6055x; 1.0749x over previous
import functools

import jax
import jax.numpy as jnp
import numpy as np
from jax import lax
from jax.experimental import pallas as pl
from jax.experimental.pallas import tpu as pltpu
from jax.experimental.pallas import tpu_sc as plsc

N = 10000
E = 160000
IN = 64
HID = 32
EDGE = 16
OUT = 128
G = 64

YW = EDGE * HID + HID
YPW = 384

NC = 2
NS = 16
NW = NC * NS
PER_W = E // NW
GROUP = 1000
NGROUP = PER_W // GROUP
SUB = 40
NSUB = GROUP // SUB
ROWS_PER_TILE = N // NS


def _sc_mesh():
    return plsc.VectorSubcoreMesh(core_axis_name="c", subcore_axis_name="s")


def _edge_body(y_hbm, src_hbm, dst_hbm, ea_hbm, zero_hbm, out_hbm,
               src_g, dst_g, ea_g, rows0, rows1, msg_g, agg_sh, sem0, sem1):
    cid = lax.axis_index("c")
    sid = lax.axis_index("s")
    wid = sid * NC + cid
    r0 = sid * ROWS_PER_TILE
    pltpu.sync_copy(zero_hbm.at[pl.ds(r0, ROWS_PER_TILE)],
                    agg_sh.at[pl.ds(r0, ROWS_PER_TILE)])
    plsc.subcore_barrier()
    base = wid * PER_W

    def compute_sub(rows_b, s):
        @plsc.parallel_loop(0, SUB, unroll=8)
        def edge_body(e):
            eg = s * SUB + e
            ea_row = ea_g[pl.ds(eg * EDGE, EDGE)]

            def ypair(k):
                word = plsc.bitcast(rows_b[e, pl.ds(k * 16, 16)],
                                    jnp.bfloat16)
                return plsc.unpack(word,
                                   format=plsc.PackFormat.INTERLEAVED)

            acc0, acc1 = ypair(EDGE)
            for k in range(EDGE):
                w = ea_row[k]
                y0, y1 = ypair(k)
                acc0 = acc0 + w * y0
                acc1 = acc1 + w * y1
            msg_g[eg, pl.ds(0, 16)] = acc0
            msg_g[eg, pl.ds(16, 16)] = acc1

    def gather_start(s, rows_b, sem_b):
        pltpu.async_copy(y_hbm.at[src_g.at[pl.ds(s * SUB, SUB)]],
                         rows_b, sem_b)

    def gather_wait(s, rows_b, sem_b):
        pltpu.make_async_copy(y_hbm.at[src_g.at[pl.ds(s * SUB, SUB)]],
                              rows_b, sem_b).wait()

    for g in range(NGROUP):
        goff = pl.multiple_of(base + g * GROUP, 8)
        goff16 = pl.multiple_of((base + g * GROUP) * EDGE, 8)
        pltpu.sync_copy(src_hbm.at[pl.ds(goff, GROUP)], src_g)
        pltpu.sync_copy(dst_hbm.at[pl.ds(goff, GROUP)], dst_g)
        pltpu.sync_copy(ea_hbm.at[pl.ds(goff16, GROUP * EDGE)], ea_g)
        gather_start(0, rows0, sem0)

        def pair_body(i, carry):
            s0 = 2 * i
            s1 = 2 * i + 1
            gather_start(s1, rows1, sem1)
            gather_wait(s0, rows0, sem0)
            compute_sub(rows0, s0)
            gather_start(s0 + 2, rows0, sem0)
            gather_wait(s1, rows1, sem1)
            compute_sub(rows1, s1)
            return carry

        lax.fori_loop(0, NSUB // 2, pair_body, 0)
        gather_wait(NSUB - 1, rows0, sem0)
        compute_sub(rows0, NSUB - 1)
        pltpu.sync_copy(msg_g, agg_sh.at[dst_g], add=True)

    plsc.subcore_barrier()
    pltpu.sync_copy(agg_sh.at[pl.ds(r0, ROWS_PER_TILE)],
                    out_hbm.at[cid, pl.ds(r0, ROWS_PER_TILE)])


@functools.cache
def _sc_edge():
    return functools.partial(
        pl.kernel,
        out_type=jax.ShapeDtypeStruct((NC, N, HID), jnp.float32),
        mesh=_sc_mesh(),
        scratch_types=[
            pltpu.VMEM((GROUP,), jnp.int32),
            pltpu.VMEM((GROUP,), jnp.int32),
            pltpu.VMEM((GROUP * EDGE,), jnp.float32),
            pltpu.VMEM((SUB, YPW), jnp.float32),
            pltpu.VMEM((SUB, YPW), jnp.float32),
            pltpu.VMEM((GROUP, HID), jnp.float32),
            pltpu.VMEM_SHARED((N, HID), jnp.float32),
            pltpu.SemaphoreType.DMA,
            pltpu.SemaphoreType.DMA,
        ],
        compiler_params=pltpu.CompilerParams(use_tc_tiling_on_sc=False,
                                             needs_layout_passes=False),
    )(_edge_body)


TN = 2000


def _y_body(x_ref, a_ref, out_ref):
    y = jnp.dot(x_ref[...], a_ref[...],
                preferred_element_type=jnp.float32).astype(jnp.bfloat16)
    lo = jax.lax.bitcast_convert_type(
        y[:, :YW // 2], jnp.uint16).astype(jnp.uint32)
    hi = jax.lax.bitcast_convert_type(
        y[:, YW // 2:], jnp.uint16).astype(jnp.uint32)
    packed = jax.lax.bitcast_convert_type(lo | (hi << 16), jnp.float32)
    out_ref[:, :YW // 2] = packed
    out_ref[:, YW // 2:] = jnp.zeros((TN, YPW - YW // 2), jnp.float32)


def _y_kernel(x, a2ext):
    return pl.pallas_call(
        _y_body,
        grid=(N // TN,),
        in_specs=[
            pl.BlockSpec((TN, IN), lambda i: (i, 0)),
            pl.BlockSpec((IN, YW), lambda i: (0, 0)),
        ],
        out_specs=pl.BlockSpec((TN, YPW), lambda i: (i, 0)),
        out_shape=jax.ShapeDtypeStruct((N, YPW), jnp.float32),
        compiler_params=pltpu.CompilerParams(
            dimension_semantics=("arbitrary",)),
    )(x, a2ext)


def _tail_body(agg_ref, x_ref, root_ref, cb_ref, g_ref, be_ref,
               sw_ref, sb_ref, tw_ref, tb_ref, batch_ref, out_ref):
    h = (agg_ref[0] + agg_ref[1]
         + jnp.dot(x_ref[...], root_ref[...], preferred_element_type=jnp.float32)
         + cb_ref[...])
    mean = jnp.mean(h, axis=0, keepdims=True)
    var = jnp.mean((h - mean) * (h - mean), axis=0, keepdims=True)
    hn = (h - mean) * lax.rsqrt(var + 1e-5) * g_ref[...] + be_ref[...]
    hr = jnp.maximum(hn, 0.0)
    s_x = jnp.dot(hr, sw_ref[...], preferred_element_type=jnp.float32) + sb_ref[...]
    t_x = jnp.dot(hr, tw_ref[...], preferred_element_type=jnp.float32) + tb_ref[...]
    s_x = jnp.clip(s_x, -30.0, 30.0)
    f_x = 1.0 / (1.0 + jnp.exp(s_x)) * jnp.tanh(t_x)
    gids = lax.broadcasted_iota(jnp.int32, (G, N), 0)
    onehot = jnp.where(gids == batch_ref[...], 1.0, 0.0)
    out_ref[...] = jnp.dot(onehot, f_x, preferred_element_type=jnp.float32)


def _tail_kernel(agg, x, root, conv_bias, bn_gamma, bn_beta,
                 s_wt, s_b, t_wt, t_b, batchi):
    return pl.pallas_call(
        _tail_body,
        out_shape=jax.ShapeDtypeStruct((G, OUT), jnp.float32),
    )(agg, x, root, conv_bias, bn_gamma, bn_beta,
      s_wt, s_b, t_wt, t_b, batchi)


def kernel(x, edge_index, edge_attr, batch, edge_batch, nn_W, nn_b, root,
           conv_bias, bn_gamma, bn_beta, s_W, s_b, t_W, t_b):
    src = edge_index[0].astype(jnp.int32)
    dst = edge_index[1].astype(jnp.int32)
    a2 = nn_W.reshape(IN, HID, EDGE).transpose(0, 2, 1).reshape(IN, EDGE * HID)
    a2ext = jnp.concatenate([a2, nn_b.reshape(IN, HID)], axis=1)
    nb = YW // HID
    lows = (np.arange(nb)[:, None] * HID + np.arange(16)[None, :]).reshape(-1)
    perm = np.concatenate([lows, lows + 16])
    a2ext = a2ext[:, perm]

    y = _y_kernel(x, a2ext)
    zeros = jnp.zeros((N, HID), jnp.float32)
    agg = _sc_edge()(y, src, dst, edge_attr.reshape(-1), zeros)

    feature = _tail_kernel(
        agg, x, root,
        conv_bias.reshape(1, HID), bn_gamma.reshape(1, HID),
        bn_beta.reshape(1, HID),
        s_W.T, s_b.reshape(1, OUT), t_W.T, t_b.reshape(1, OUT),
        batch.astype(jnp.int32).reshape(1, N))
    return feature


# speedup vs baseline: 6.9171x; 1.0472x over previous
import functools

import jax
import jax.numpy as jnp
import numpy as np
from jax import lax
from jax.experimental import pallas as pl
from jax.experimental.pallas import tpu as pltpu
from jax.experimental.pallas import tpu_sc as plsc

N = 10000
E = 160000
IN = 64
HID = 32
EDGE = 16
OUT = 128
G = 64

YW = EDGE * HID + HID
YPW = YW // 2

NC = 2
NS = 16
NW = NC * NS
PER_W = E // NW
GROUP = 1000
NGROUP = PER_W // GROUP
SUB = 40
NSUB = GROUP // SUB
ROWS_PER_TILE = N // NS


def _sc_mesh():
    return plsc.VectorSubcoreMesh(core_axis_name="c", subcore_axis_name="s")


def _edge_body(y_hbm, src_hbm, dst_hbm, ea_hbm, zero_hbm, out_hbm,
               src_g, dst_g, ea_g, rows0, rows1, msg_g, agg_sh, sem0, sem1):
    cid = lax.axis_index("c")
    sid = lax.axis_index("s")
    wid = sid * NC + cid
    r0 = sid * ROWS_PER_TILE
    pltpu.sync_copy(zero_hbm.at[pl.ds(r0, ROWS_PER_TILE)],
                    agg_sh.at[pl.ds(r0, ROWS_PER_TILE)])
    plsc.subcore_barrier()
    base = wid * PER_W

    def compute_sub(rows_b, s):
        @plsc.parallel_loop(0, SUB, unroll=4)
        def edge_body(e):
            eg = s * SUB + e
            ea_row = ea_g[pl.ds(eg * EDGE, EDGE)]

            def ypair(k):
                word = plsc.bitcast(rows_b[e, pl.ds(k * 16, 16)],
                                    jnp.bfloat16)
                return plsc.unpack(word,
                                   format=plsc.PackFormat.INTERLEAVED)

            acc0, acc1 = ypair(EDGE)
            for k in range(EDGE):
                w = ea_row[k]
                y0, y1 = ypair(k)
                acc0 = acc0 + w * y0
                acc1 = acc1 + w * y1
            msg_g[eg, pl.ds(0, 16)] = acc0
            msg_g[eg, pl.ds(16, 16)] = acc1

    def gather_start(s, rows_b, sem_b):
        pltpu.async_copy(y_hbm.at[src_g.at[pl.ds(s * SUB, SUB)]],
                         rows_b, sem_b)

    def gather_wait(s, rows_b, sem_b):
        pltpu.make_async_copy(y_hbm.at[src_g.at[pl.ds(s * SUB, SUB)]],
                              rows_b, sem_b).wait()

    for g in range(NGROUP):
        goff = pl.multiple_of(base + g * GROUP, 8)
        goff16 = pl.multiple_of((base + g * GROUP) * EDGE, 8)
        pltpu.sync_copy(src_hbm.at[pl.ds(goff, GROUP)], src_g)
        pltpu.sync_copy(dst_hbm.at[pl.ds(goff, GROUP)], dst_g)
        pltpu.sync_copy(ea_hbm.at[pl.ds(goff16, GROUP * EDGE)], ea_g)
        gather_start(0, rows0, sem0)

        def pair_body(i, carry):
            s0 = 2 * i
            s1 = 2 * i + 1
            gather_start(s1, rows1, sem1)
            gather_wait(s0, rows0, sem0)
            compute_sub(rows0, s0)
            gather_start(s0 + 2, rows0, sem0)
            gather_wait(s1, rows1, sem1)
            compute_sub(rows1, s1)
            return carry

        lax.fori_loop(0, NSUB // 2, pair_body, 0)
        gather_wait(NSUB - 1, rows0, sem0)
        compute_sub(rows0, NSUB - 1)
        pltpu.sync_copy(msg_g, agg_sh.at[dst_g], add=True)

    plsc.subcore_barrier()
    pltpu.sync_copy(agg_sh.at[pl.ds(r0, ROWS_PER_TILE)],
                    out_hbm.at[cid, pl.ds(r0, ROWS_PER_TILE)])


@functools.cache
def _sc_edge():
    return functools.partial(
        pl.kernel,
        out_type=jax.ShapeDtypeStruct((NC, N, HID), jnp.float32),
        mesh=_sc_mesh(),
        scratch_types=[
            pltpu.VMEM((GROUP,), jnp.int32),
            pltpu.VMEM((GROUP,), jnp.int32),
            pltpu.VMEM((GROUP * EDGE,), jnp.float32),
            pltpu.VMEM((SUB, YPW), jnp.float32),
            pltpu.VMEM((SUB, YPW), jnp.float32),
            pltpu.VMEM((GROUP, HID), jnp.float32),
            pltpu.VMEM_SHARED((N, HID), jnp.float32),
            pltpu.SemaphoreType.DMA,
            pltpu.SemaphoreType.DMA,
        ],
        compiler_params=pltpu.CompilerParams(use_tc_tiling_on_sc=False,
                                             needs_layout_passes=False),
    )(_edge_body)


TN = 2000


def _y_body(x_ref, a_ref, out_ref):
    y = jnp.dot(x_ref[...], a_ref[...],
                preferred_element_type=jnp.float32).astype(jnp.bfloat16)
    lo = jax.lax.bitcast_convert_type(
        y[:, :YW // 2], jnp.uint16).astype(jnp.uint32)
    hi = jax.lax.bitcast_convert_type(
        y[:, YW // 2:], jnp.uint16).astype(jnp.uint32)
    out_ref[...] = jax.lax.bitcast_convert_type(lo | (hi << 16), jnp.float32)


def _y_kernel(x, a2ext):
    return pl.pallas_call(
        _y_body,
        grid=(N // TN,),
        in_specs=[
            pl.BlockSpec((TN, IN), lambda i: (i, 0)),
            pl.BlockSpec((IN, YW), lambda i: (0, 0)),
        ],
        out_specs=pl.BlockSpec((TN, YPW), lambda i: (i, 0)),
        out_shape=jax.ShapeDtypeStruct((N, YPW), jnp.float32),
        compiler_params=pltpu.CompilerParams(
            dimension_semantics=("arbitrary",)),
    )(x, a2ext)


def _tail_body(agg_ref, x_ref, root_ref, cb_ref, g_ref, be_ref,
               sw_ref, sb_ref, tw_ref, tb_ref, batch_ref, out_ref):
    h = (agg_ref[0] + agg_ref[1]
         + jnp.dot(x_ref[...], root_ref[...], preferred_element_type=jnp.float32)
         + cb_ref[...])
    mean = jnp.mean(h, axis=0, keepdims=True)
    var = jnp.mean((h - mean) * (h - mean), axis=0, keepdims=True)
    hn = (h - mean) * lax.rsqrt(var + 1e-5) * g_ref[...] + be_ref[...]
    hr = jnp.maximum(hn, 0.0)
    s_x = jnp.dot(hr, sw_ref[...], preferred_element_type=jnp.float32) + sb_ref[...]
    t_x = jnp.dot(hr, tw_ref[...], preferred_element_type=jnp.float32) + tb_ref[...]
    s_x = jnp.clip(s_x, -30.0, 30.0)
    f_x = 1.0 / (1.0 + jnp.exp(s_x)) * jnp.tanh(t_x)
    gids = lax.broadcasted_iota(jnp.int32, (G, N), 0)
    onehot = jnp.where(gids == batch_ref[...], 1.0, 0.0)
    out_ref[...] = jnp.dot(onehot, f_x, preferred_element_type=jnp.float32)


def _tail_kernel(agg, x, root, conv_bias, bn_gamma, bn_beta,
                 s_wt, s_b, t_wt, t_b, batchi):
    return pl.pallas_call(
        _tail_body,
        out_shape=jax.ShapeDtypeStruct((G, OUT), jnp.float32),
    )(agg, x, root, conv_bias, bn_gamma, bn_beta,
      s_wt, s_b, t_wt, t_b, batchi)


def kernel(x, edge_index, edge_attr, batch, edge_batch, nn_W, nn_b, root,
           conv_bias, bn_gamma, bn_beta, s_W, s_b, t_W, t_b):
    src = edge_index[0].astype(jnp.int32)
    dst = edge_index[1].astype(jnp.int32)
    a2 = nn_W.reshape(IN, HID, EDGE).transpose(0, 2, 1).reshape(IN, EDGE * HID)
    a2ext = jnp.concatenate([a2, nn_b.reshape(IN, HID)], axis=1)
    nb = YW // HID
    lows = (np.arange(nb)[:, None] * HID + np.arange(16)[None, :]).reshape(-1)
    perm = np.concatenate([lows, lows + 16])
    a2ext = a2ext[:, perm]

    y = _y_kernel(x, a2ext)
    zeros = jnp.zeros((N, HID), jnp.float32)
    agg = _sc_edge()(y, src, dst, edge_attr.reshape(-1), zeros)

    feature = _tail_kernel(
        agg, x, root,
        conv_bias.reshape(1, HID), bn_gamma.reshape(1, HID),
        bn_beta.reshape(1, HID),
        s_W.T, s_b.reshape(1, OUT), t_W.T, t_b.reshape(1, OUT),
        batch.astype(jnp.int32).reshape(1, N))
    return feature
